# baseline (speedup 1.0000x reference)
import functools

import jax
import jax.numpy as jnp
from jax import lax
from jax.experimental import pallas as pl
from jax.experimental.pallas import tpu as pltpu
from jax.experimental.pallas import tpu_sc as plsc


def _make_sc_gather(n_main, n_item, D):
    info = plsc.get_sparse_core_info()
    NC, NS = info.num_cores, info.num_subcores
    NW = NC * NS
    assert n_main % NW == 0 and n_item % NW == 0
    pw = n_main // NW
    C = 544
    assert pw % C == 0
    nchunk = pw // C
    piw = n_item // NW
    assert piw % 8 == 0

    mesh = plsc.VectorSubcoreMesh(core_axis_name="c", subcore_axis_name="s")

    @functools.partial(
        pl.kernel,
        mesh=mesh,
        compiler_params=pltpu.CompilerParams(use_tc_tiling_on_sc=False),
        out_type=[
            jax.ShapeDtypeStruct((n_main, D), jnp.float32),
            jax.ShapeDtypeStruct((n_main, D), jnp.float32),
            jax.ShapeDtypeStruct((n_item, D), jnp.float32),
        ],
        scratch_types=[
            pltpu.VMEM((C,), jnp.int32),
            pltpu.VMEM((C, D), jnp.float32),
            pltpu.VMEM((C, D), jnp.float32),
            pltpu.VMEM((piw,), jnp.int32),
            pltpu.VMEM((piw, D), jnp.float32),
            pltpu.SemaphoreType.DMA,
            pltpu.SemaphoreType.DMA,
        ],
    )
    def sc_gather(idx_hbm, umem_hbm, uout_hbm, iidx_hbm, imem_hbm,
                  gmem_hbm, gout_hbm, gitem_hbm,
                  idx_v, rows_a, rows_b, iidx_v, irows_v, sem_a, sem_b):
        wid = lax.axis_index("s") * NC + lax.axis_index("c")
        base = wid * pw

        def chunk(c, carry):
            off = base + c * C
            pltpu.sync_copy(idx_hbm.at[pl.ds(off, C)], idx_v)
            cp_a = pltpu.async_copy(umem_hbm.at[idx_v], rows_a, sem_a)
            cp_b = pltpu.async_copy(uout_hbm.at[idx_v], rows_b, sem_b)
            cp_a.wait()
            pltpu.sync_copy(rows_a, gmem_hbm.at[pl.ds(off, C)])
            cp_b.wait()
            pltpu.sync_copy(rows_b, gout_hbm.at[pl.ds(off, C)])
            return carry

        lax.fori_loop(0, nchunk, chunk, 0)

        ibase = wid * piw
        pltpu.sync_copy(iidx_hbm.at[pl.ds(ibase, piw)], iidx_v)
        pltpu.async_copy(imem_hbm.at[iidx_v], irows_v, sem_a).wait()
        pltpu.sync_copy(irows_v, gitem_hbm.at[pl.ds(ibase, piw)])

    return sc_gather


def _dense_body(cu_ref, ci_ref, mem_ref, nout_ref, whop_ref, bhop_ref,
                wout_ref, bout_ref, w1_ref, score_ref):
    cu = cu_ref[...]
    ci = ci_ref[...]
    mem = mem_ref[...]
    nout = nout_ref[...]
    q = cu + ci

    def attend(query):
        s = jnp.sum(query[:, None, :] * mem, axis=2)
        s = s - jnp.max(s, axis=-1, keepdims=True)
        e = jnp.exp(s)
        a = e / jnp.sum(e, axis=-1, keepdims=True)
        return jnp.sum(a[:, :, None] * nout, axis=1)

    w0 = attend(q)
    q1 = jnp.dot(q, whop_ref[...], preferred_element_type=jnp.float32)
    q1 = jnp.maximum(q1 + bhop_ref[...] + w0, 0.0)
    w1 = attend(q1)

    aa = cu * ci
    d = cu.shape[1]
    z = (jnp.dot(aa, wout_ref[:d], preferred_element_type=jnp.float32)
         + jnp.dot(w1, wout_ref[d:], preferred_element_type=jnp.float32)
         + bout_ref[...])
    score = jnp.maximum(jnp.sum(z * w1_ref[...], axis=1, keepdims=True), 0.0)
    score_ref[...] = score


def _dense(cu, ci, nmem, nout, W_hop, b_hop, W_out, b_out, W_1,
           interpret=False):
    B, D = cu.shape
    L = nmem.shape[1]
    BB = 256
    grid = (B // BB,)
    return pl.pallas_call(
        _dense_body,
        grid=grid,
        in_specs=[
            pl.BlockSpec((BB, D), lambda i: (i, 0)),
            pl.BlockSpec((BB, D), lambda i: (i, 0)),
            pl.BlockSpec((BB, L, D), lambda i: (i, 0, 0)),
            pl.BlockSpec((BB, L, D), lambda i: (i, 0, 0)),
            pl.BlockSpec((D, D), lambda i: (0, 0)),
            pl.BlockSpec((1, D), lambda i: (0, 0)),
            pl.BlockSpec((2 * D, D), lambda i: (0, 0)),
            pl.BlockSpec((1, D), lambda i: (0, 0)),
            pl.BlockSpec((1, D), lambda i: (0, 0)),
        ],
        out_specs=pl.BlockSpec((BB, 1), lambda i: (i, 0)),
        out_shape=jax.ShapeDtypeStruct((B, 1), jnp.float32),
        interpret=interpret,
    )(cu, ci, nmem, nout, W_hop, b_hop, W_out, b_out, W_1)


def kernel(input_users, input_items, input_items_negative, input_neighborhoods,
           input_neighborhood_lengths, input_neighborhoods_negative,
           input_neighborhood_lengths_negative, user_memory, user_output,
           item_memory, W_hop, b_hop, W_out, b_out, W_1):
    B, L = input_neighborhoods.shape
    D = user_memory.shape[1]

    neigh = input_neighborhoods.astype(jnp.int32).reshape(-1)
    idx_all = jnp.concatenate([neigh, input_users.astype(jnp.int32)])
    n_main = B * L + B
    item_idx = input_items.astype(jnp.int32)

    sc_gather = _make_sc_gather(n_main, B, D)
    gmem, gout, gitem = sc_gather(idx_all, user_memory, user_output,
                                  item_idx, item_memory)

    cur_user = gmem[B * L:]
    cur_item = gitem
    nmem = gmem[:B * L].reshape(B, L, D)
    nout = gout[:B * L].reshape(B, L, D)

    return _dense(cur_user, cur_item, nmem, nout, W_hop,
                  b_hop.reshape(1, D), W_out, b_out.reshape(1, D),
                  W_1.reshape(1, D))


# speedup vs baseline: 1.1992x; 1.1992x over previous
import functools

import jax
import jax.numpy as jnp
from jax import lax
from jax.experimental import pallas as pl
from jax.experimental.pallas import tpu as pltpu
from jax.experimental.pallas import tpu_sc as plsc


def _make_sc_gather(n_main, n_cur, D):
    info = plsc.get_sparse_core_info()
    NC, NS = info.num_cores, info.num_subcores
    NW = NC * NS
    assert n_main % NW == 0 and n_cur % NW == 0
    pw = n_main // NW
    C = 640
    assert pw % C == 0
    nchunk = pw // C
    piw = n_cur // NW
    assert piw % 8 == 0

    mesh = plsc.VectorSubcoreMesh(core_axis_name="c", subcore_axis_name="s")

    @functools.partial(
        pl.kernel,
        mesh=mesh,
        compiler_params=pltpu.CompilerParams(use_tc_tiling_on_sc=False),
        out_type=[
            jax.ShapeDtypeStruct((n_main, 2 * D), jnp.float32),
            jax.ShapeDtypeStruct((n_cur, 2 * D), jnp.float32),
        ],
        scratch_types=[
            pltpu.VMEM((C,), jnp.int32),
            pltpu.VMEM((C, D), jnp.float32),
            pltpu.VMEM((C, D), jnp.float32),
            pltpu.VMEM((piw,), jnp.int32),
            pltpu.VMEM((piw,), jnp.int32),
            pltpu.VMEM((piw, D), jnp.float32),
            pltpu.VMEM((piw, D), jnp.float32),
            pltpu.SemaphoreType.DMA,
            pltpu.SemaphoreType.DMA,
        ],
    )
    def sc_gather(idx_hbm, umem_hbm, uout_hbm, uidx_hbm, iidx_hbm, imem_hbm,
                  big_hbm, small_hbm,
                  idx_v, rows_a, rows_b, uidx_v, iidx_v, crows_a, crows_b,
                  sem_a, sem_b):
        wid = lax.axis_index("s") * NC + lax.axis_index("c")
        base = wid * pw

        def chunk(c, carry):
            off = base + c * C
            pltpu.sync_copy(idx_hbm.at[pl.ds(off, C)], idx_v)
            cp_a = pltpu.async_copy(umem_hbm.at[idx_v], rows_a, sem_a)
            cp_b = pltpu.async_copy(uout_hbm.at[idx_v], rows_b, sem_b)
            cp_a.wait()
            pltpu.sync_copy(rows_a, big_hbm.at[pl.ds(off, C), pl.ds(0, D)])
            cp_b.wait()
            pltpu.sync_copy(rows_b, big_hbm.at[pl.ds(off, C), pl.ds(D, D)])
            return carry

        lax.fori_loop(0, nchunk, chunk, 0)

        ibase = wid * piw
        pltpu.sync_copy(uidx_hbm.at[pl.ds(ibase, piw)], uidx_v)
        pltpu.sync_copy(iidx_hbm.at[pl.ds(ibase, piw)], iidx_v)
        cp_a = pltpu.async_copy(umem_hbm.at[uidx_v], crows_a, sem_a)
        cp_b = pltpu.async_copy(imem_hbm.at[iidx_v], crows_b, sem_b)
        cp_a.wait()
        pltpu.sync_copy(crows_a, small_hbm.at[pl.ds(ibase, piw), pl.ds(0, D)])
        cp_b.wait()
        pltpu.sync_copy(crows_b, small_hbm.at[pl.ds(ibase, piw), pl.ds(D, D)])

    return sc_gather


def _dense_body(cur_ref, big_ref, whop_ref, bhop_ref,
                wout_ref, bout_ref, w1_ref, score_ref):
    d = whop_ref.shape[0]
    cu = cur_ref[:, :d]
    ci = cur_ref[:, d:]
    mem = big_ref[:, :, :d]
    nout = big_ref[:, :, d:]
    q = cu + ci

    def attend(query):
        s = jnp.sum(query[:, None, :] * mem, axis=2)
        s = s - jnp.max(s, axis=-1, keepdims=True)
        e = jnp.exp(s)
        a = e / jnp.sum(e, axis=-1, keepdims=True)
        return jnp.sum(a[:, :, None] * nout, axis=1)

    w0 = attend(q)
    q1 = jnp.dot(q, whop_ref[...], preferred_element_type=jnp.float32)
    q1 = jnp.maximum(q1 + bhop_ref[...] + w0, 0.0)
    w1 = attend(q1)

    aa = cu * ci
    z = (jnp.dot(aa, wout_ref[:d], preferred_element_type=jnp.float32)
         + jnp.dot(w1, wout_ref[d:], preferred_element_type=jnp.float32)
         + bout_ref[...])
    score = jnp.maximum(jnp.sum(z * w1_ref[...], axis=1, keepdims=True), 0.0)
    score_ref[...] = score


def _dense(cur, big, W_hop, b_hop, W_out, b_out, W_1, interpret=False):
    B = cur.shape[0]
    L = big.shape[1]
    D = W_hop.shape[0]
    BB = 256
    grid = (B // BB,)
    return pl.pallas_call(
        _dense_body,
        grid=grid,
        in_specs=[
            pl.BlockSpec((BB, 2 * D), lambda i: (i, 0)),
            pl.BlockSpec((BB, L, 2 * D), lambda i: (i, 0, 0)),
            pl.BlockSpec((D, D), lambda i: (0, 0)),
            pl.BlockSpec((1, D), lambda i: (0, 0)),
            pl.BlockSpec((2 * D, D), lambda i: (0, 0)),
            pl.BlockSpec((1, D), lambda i: (0, 0)),
            pl.BlockSpec((1, D), lambda i: (0, 0)),
        ],
        out_specs=pl.BlockSpec((BB, 1), lambda i: (i, 0)),
        out_shape=jax.ShapeDtypeStruct((B, 1), jnp.float32),
        interpret=interpret,
    )(cur, big, W_hop, b_hop, W_out, b_out, W_1)


def kernel(input_users, input_items, input_items_negative, input_neighborhoods,
           input_neighborhood_lengths, input_neighborhoods_negative,
           input_neighborhood_lengths_negative, user_memory, user_output,
           item_memory, W_hop, b_hop, W_out, b_out, W_1):
    B, L = input_neighborhoods.shape
    D = user_memory.shape[1]

    neigh = input_neighborhoods.astype(jnp.int32).reshape(-1)
    uidx = input_users.astype(jnp.int32)
    iidx = input_items.astype(jnp.int32)

    sc_gather = _make_sc_gather(B * L, B, D)
    big, small = sc_gather(neigh, user_memory, user_output,
                           uidx, iidx, item_memory)

    big3 = big.reshape(B, L, 2 * D)

    return _dense(small, big3, W_hop,
                  b_hop.reshape(1, D), W_out, b_out.reshape(1, D),
                  W_1.reshape(1, D))
